# Initial kernel scaffold; baseline (speedup 1.0000x reference)
import jax
import jax.numpy as jnp
from jax.experimental import pallas as pl


def kernel(x, edge_index, edge_weight):
    raise NotImplementedError("write your pallas kernel here")


# speedup vs baseline: 2.6658x; 2.6658x over previous
import functools

import jax
import jax.numpy as jnp
from jax import lax
from jax.experimental import pallas as pl
from jax.experimental.pallas import tpu as pltpu
from jax.experimental.pallas import tpu_sc as plsc

N_NODES = 10000
N_EDGES = 160000
D_FEAT = 256
D_HALF = D_FEAT // 2

NUM_CORES = 2
NUM_SUBCORES = 16
LANES = 16

EDGES_PER_TILE = N_EDGES // NUM_SUBCORES
CHUNK = 80
NUM_CHUNKS = EDGES_PER_TILE // CHUNK
ROW_BASE = 624
BLK_ROWS = 16
D_BLKS = D_HALF // LANES


def _gcn_sc_kernel(x_hbm, row_hbm, col_hbm, w_hbm, out_hbm,
                   acc, col_v, row_v, w_v, rows_v, zeros_v, sem):
    c = lax.axis_index("c")
    s = lax.axis_index("s")

    zvec = jnp.zeros((LANES,), jnp.float32)
    def zero_body(i, _):
        for d in range(D_BLKS):
            zeros_v[i, pl.ds(d * LANES, LANES)] = zvec
        return 0
    lax.fori_loop(0, BLK_ROWS, zero_body, 0)
    base_row = s * ROW_BASE
    nblk = jnp.where(s == NUM_SUBCORES - 1, 40, 39)
    def zero_copy(z, _):
        pltpu.sync_copy(zeros_v, acc.at[pl.ds(base_row + z * BLK_ROWS,
                                              BLK_ROWS)])
        return 0
    lax.fori_loop(0, nblk, zero_copy, 0)
    plsc.subcore_barrier()

    edge_base = s * EDGES_PER_TILE

    def chunk_body(g, _):
        base = edge_base + g * CHUNK
        pltpu.sync_copy(col_hbm.at[pl.ds(base, CHUNK)], col_v)
        pltpu.sync_copy(row_hbm.at[pl.ds(base, CHUNK)], row_v)
        pltpu.sync_copy(w_hbm.at[pl.ds(base, CHUNK)], w_v)
        pltpu.async_copy(x_hbm.at[c].at[col_v], rows_v, sem).wait()

        def group_body(gi, _):
            w16 = w_v[pl.ds(gi * LANES, LANES)]
            for j in range(LANES):
                e = gi * LANES + j
                wscal = w16[j]
                for d in range(D_BLKS):
                    blk = rows_v[e, pl.ds(d * LANES, LANES)]
                    rows_v[e, pl.ds(d * LANES, LANES)] = blk * wscal
            return 0
        lax.fori_loop(0, CHUNK // LANES, group_body, 0)

        pltpu.sync_copy(rows_v, acc.at[row_v], add=True)
        return 0

    lax.fori_loop(0, NUM_CHUNKS, chunk_body, 0)
    plsc.subcore_barrier()

    def wb_copy(z, _):
        off = base_row + z * BLK_ROWS
        pltpu.sync_copy(acc.at[pl.ds(off, BLK_ROWS)],
                        out_hbm.at[c].at[pl.ds(off, BLK_ROWS)])
        return 0
    lax.fori_loop(0, nblk, wb_copy, 0)


@jax.jit
def _gcn(x, row, col, w):
    x2 = jnp.transpose(x.reshape(N_NODES, NUM_CORES, D_HALF), (1, 0, 2))
    mesh = plsc.VectorSubcoreMesh(core_axis_name="c", subcore_axis_name="s")
    out2 = pl.kernel(
        _gcn_sc_kernel,
        mesh=mesh,
        out_type=jax.ShapeDtypeStruct((NUM_CORES, N_NODES, D_HALF),
                                      jnp.float32),
        scratch_types=[
            pltpu.VMEM_SHARED((N_NODES, D_HALF), jnp.float32),
            pltpu.VMEM((CHUNK,), jnp.int32),
            pltpu.VMEM((CHUNK,), jnp.int32),
            pltpu.VMEM((CHUNK,), jnp.float32),
            pltpu.VMEM((CHUNK, D_HALF), jnp.float32),
            pltpu.VMEM((BLK_ROWS, D_HALF), jnp.float32),
            pltpu.SemaphoreType.DMA,
        ],
    )(x2, row, col, w)
    return jnp.transpose(out2, (1, 0, 2)).reshape(N_NODES, D_FEAT)


def kernel(x, edge_index, edge_weight):
    row = edge_index[0].astype(jnp.int32)
    col = edge_index[1].astype(jnp.int32)
    return _gcn(x, row, col, edge_weight)


# speedup vs baseline: 7.5773x; 2.8424x over previous
import functools

import jax
import jax.numpy as jnp
from jax import lax
from jax.experimental import pallas as pl
from jax.experimental.pallas import tpu as pltpu
from jax.experimental.pallas import tpu_sc as plsc

N_NODES = 10000
N_EDGES = 160000
D_FEAT = 256
D_HALF = D_FEAT // 2

NUM_CORES = 2
NUM_SUBCORES = 16
LANES = 16

EDGES_PER_TILE = N_EDGES // NUM_SUBCORES
CHUNK = 80
NUM_CHUNKS = EDGES_PER_TILE // CHUNK
GROUPS = CHUNK // LANES
ROW_BASE = 624
BLK_ROWS = 16
D_BLKS = D_HALF // LANES

R_ROWS = 4
R_IDX = 8
UNROLL = 8
MAIN_CHUNKS = 120
GATHER_AHEAD = 2
IDX_AHEAD = 6


def _gcn_sc_kernel(x_hbm, row_hbm, col_hbm, w_hbm, out_hbm,
                   acc, row_sm, col_sm, w_sm,
                   rows0, rows1, rows2, rows3,
                   gs0, gs1, gs2, gs3, ss0, ss1, ss2, ss3,
                   is0, is1, is2, is3, is4, is5, is6, is7):
    c = lax.axis_index("c")
    s = lax.axis_index("s")
    rows = [rows0, rows1, rows2, rows3]
    gsem = [gs0, gs1, gs2, gs3]
    ssem = [ss0, ss1, ss2, ss3]
    isem = [is0, is1, is2, is3, is4, is5, is6, is7]
    ebase = s * EDGES_PER_TILE

    def idx_issue(g, q):
        base = ebase + g * CHUNK
        pltpu.async_copy(row_hbm.at[pl.ds(base, CHUNK)], row_sm.at[q],
                         isem[q])
        pltpu.async_copy(col_hbm.at[pl.ds(base, CHUNK)], col_sm.at[q],
                         isem[q])
        pltpu.async_copy(w_hbm.at[pl.ds(base, CHUNK)], w_sm.at[q], isem[q])

    def idx_wait(q):
        pltpu.make_async_copy(row_hbm.at[pl.ds(0, CHUNK)], row_sm.at[q],
                              isem[q]).wait()
        pltpu.make_async_copy(col_hbm.at[pl.ds(0, CHUNK)], col_sm.at[q],
                              isem[q]).wait()
        pltpu.make_async_copy(w_hbm.at[pl.ds(0, CHUNK)], w_sm.at[q],
                              isem[q]).wait()

    def transform(q):
        for gi in range(GROUPS):
            v = col_sm[q, pl.ds(gi * LANES, LANES)]
            col_sm[q, pl.ds(gi * LANES, LANES)] = v * 2 + c

    def gather_issue(q, p):
        pltpu.async_copy(x_hbm.at[col_sm.at[q]], rows[p], gsem[p])

    def gather_wait(q, p):
        pltpu.make_async_copy(x_hbm.at[col_sm.at[q]], rows[p],
                              gsem[p]).wait()

    def scale(q, p):
        rp = rows[p]
        def group_body(gi, _):
            w16 = w_sm[q, pl.ds(gi * LANES, LANES)]
            for j in range(LANES):
                e = gi * LANES + j
                wscal = w16[j]
                for d in range(D_BLKS):
                    blk = rp[e, pl.ds(d * LANES, LANES)]
                    rp[e, pl.ds(d * LANES, LANES)] = blk * wscal
            return 0
        lax.fori_loop(0, GROUPS, group_body, 0)

    def scatter_issue(q, p):
        pltpu.async_copy(rows[p], acc.at[row_sm.at[q]], ssem[p], add=True)

    def scatter_wait(q, p):
        pltpu.make_async_copy(rows[p], acc.at[row_sm.at[q]],
                              ssem[p]).wait()

    for g in range(IDX_AHEAD):
        idx_issue(g, g % R_IDX)

    zvec = jnp.zeros((LANES,), jnp.float32)
    for i in range(BLK_ROWS):
        for d in range(D_BLKS):
            rows0[i, pl.ds(d * LANES, LANES)] = zvec
    base_row = s * ROW_BASE
    nblk = jnp.where(s == NUM_SUBCORES - 1, 40, 39)
    def zero_copy(z, _):
        pltpu.sync_copy(rows0.at[pl.ds(0, BLK_ROWS)],
                        acc.at[pl.ds(base_row + z * BLK_ROWS, BLK_ROWS)])
        return 0
    lax.fori_loop(0, nblk, zero_copy, 0)

    for g in range(GATHER_AHEAD):
        idx_wait(g % R_IDX)
        transform(g % R_IDX)
        gather_issue(g % R_IDX, g % R_ROWS)
    plsc.subcore_barrier()

    def emit_chunk(h, k, static_tail):
        p = k % R_ROWS
        q = k % R_IDX
        pn = (k + GATHER_AHEAD) % R_ROWS
        qn = (k + GATHER_AHEAD) % R_IDX
        qi = (k + IDX_AHEAD) % R_IDX
        if static_tail:
            if h >= GATHER_AHEAD:
                scatter_wait(qn, pn)
        else:
            @pl.when(h >= GATHER_AHEAD)
            def _():
                scatter_wait(qn, pn)
        if static_tail:
            if h + IDX_AHEAD < NUM_CHUNKS:
                idx_issue(h + IDX_AHEAD, qi)
            if h + GATHER_AHEAD < NUM_CHUNKS:
                idx_wait(qn)
                transform(qn)
                gather_issue(qn, pn)
        else:
            @pl.when(h + IDX_AHEAD < NUM_CHUNKS)
            def _():
                idx_issue(h + IDX_AHEAD, qi)
            idx_wait(qn)
            transform(qn)
            gather_issue(qn, pn)
        gather_wait(q, p)
        scale(q, p)
        scatter_issue(q, p)

    def main_body(i, _):
        for k in range(UNROLL):
            emit_chunk(i * UNROLL + k, k, False)
        return 0
    lax.fori_loop(0, MAIN_CHUNKS // UNROLL, main_body, 0)

    for h in range(MAIN_CHUNKS, NUM_CHUNKS):
        emit_chunk(h, h % UNROLL, True)

    scatter_wait((NUM_CHUNKS - 2) % R_IDX, (NUM_CHUNKS - 2) % R_ROWS)
    scatter_wait((NUM_CHUNKS - 1) % R_IDX, (NUM_CHUNKS - 1) % R_ROWS)
    plsc.subcore_barrier()

    col_off = pl.multiple_of(c * D_HALF, D_HALF)
    def wb_copy(z, _):
        off = base_row + z * BLK_ROWS
        pltpu.sync_copy(acc.at[pl.ds(off, BLK_ROWS)],
                        out_hbm.at[pl.ds(off, BLK_ROWS),
                                   pl.ds(col_off, D_HALF)])
        return 0
    lax.fori_loop(0, nblk, wb_copy, 0)


@jax.jit
def _gcn(x, row, col, w):
    x2 = x.reshape(N_NODES * NUM_CORES, D_HALF)
    mesh = plsc.VectorSubcoreMesh(core_axis_name="c", subcore_axis_name="s")
    dma = pltpu.SemaphoreType.DMA
    return pl.kernel(
        _gcn_sc_kernel,
        mesh=mesh,
        out_type=jax.ShapeDtypeStruct((N_NODES, D_FEAT), jnp.float32),
        scratch_types=[
            pltpu.VMEM_SHARED((N_NODES, D_HALF), jnp.float32),
            pltpu.VMEM((R_IDX, CHUNK), jnp.int32),
            pltpu.VMEM((R_IDX, CHUNK), jnp.int32),
            pltpu.VMEM((R_IDX, CHUNK), jnp.float32),
            pltpu.VMEM((CHUNK, D_HALF), jnp.float32),
            pltpu.VMEM((CHUNK, D_HALF), jnp.float32),
            pltpu.VMEM((CHUNK, D_HALF), jnp.float32),
            pltpu.VMEM((CHUNK, D_HALF), jnp.float32),
        ] + [dma] * 16,
    )(x2, row, col, w)


def kernel(x, edge_index, edge_weight):
    row = edge_index[0].astype(jnp.int32)
    col = edge_index[1].astype(jnp.int32)
    return _gcn(x, row, col, edge_weight)
